# Initial kernel scaffold; baseline (speedup 1.0000x reference)
import jax
import jax.numpy as jnp
from jax.experimental import pallas as pl


def kernel(token_ids, weight):
    raise NotImplementedError("write your pallas kernel here")


# speedup vs baseline: 2.9789x; 2.9789x over previous
import functools

import jax
import jax.numpy as jnp
from jax import lax
from jax.experimental import pallas as pl
from jax.experimental.pallas import tpu as pltpu
from jax.experimental.pallas import tpu_sc as plsc

EMBED_DIM = 128
TOKENS = 4096 * 50
NUM_CORES = 2
NUM_SUBCORES = 16
NUM_WORKERS = NUM_CORES * NUM_SUBCORES
ROWS_PER_WORKER = TOKENS // NUM_WORKERS
CHUNK = 128
CHUNKS_PER_WORKER = ROWS_PER_WORKER // CHUNK

_mesh = plsc.VectorSubcoreMesh(core_axis_name="c", subcore_axis_name="s")


@functools.partial(
    pl.kernel,
    out_type=jax.ShapeDtypeStruct((TOKENS, EMBED_DIM), jnp.float32),
    mesh=_mesh,
    scratch_types=[
        pltpu.VMEM((CHUNKS_PER_WORKER, CHUNK), jnp.int32),
        pltpu.VMEM((CHUNK, EMBED_DIM), jnp.float32),
        pltpu.SemaphoreType.DMA,
    ],
)
def _embedding_gather(idx_hbm, table_hbm, out_hbm, idx_v, rows_v, sem):
    wid = lax.axis_index("c") * NUM_SUBCORES + lax.axis_index("s")
    pltpu.sync_copy(idx_hbm.at[wid], idx_v)

    def body(j, carry):
        pltpu.async_copy(table_hbm.at[idx_v.at[j]], rows_v, sem).wait()
        base = wid * ROWS_PER_WORKER + j * CHUNK
        pltpu.sync_copy(rows_v, out_hbm.at[pl.ds(base, CHUNK)])
        return carry

    lax.fori_loop(0, CHUNKS_PER_WORKER, body, 0)


def kernel(token_ids, weight):
    flat = token_ids.reshape(-1).astype(jnp.int32)
    idx = flat.reshape(NUM_WORKERS, CHUNKS_PER_WORKER, CHUNK)
    out = _embedding_gather(idx, weight)
    return out.reshape(token_ids.shape + (weight.shape[1],))


# speedup vs baseline: 3.3550x; 1.1263x over previous
import functools

import jax
import jax.numpy as jnp
from jax import lax
from jax.experimental import pallas as pl
from jax.experimental.pallas import tpu as pltpu
from jax.experimental.pallas import tpu_sc as plsc

EMBED_DIM = 128
TOKENS = 4096 * 50
NUM_CORES = 2
NUM_SUBCORES = 16
NUM_WORKERS = NUM_CORES * NUM_SUBCORES
ROWS_PER_WORKER = TOKENS // NUM_WORKERS
CHUNK = 128
CHUNKS_PER_WORKER = ROWS_PER_WORKER // CHUNK
NBUF = 5
LOOK = 3
ROUNDS = CHUNKS_PER_WORKER // NBUF

_mesh = plsc.VectorSubcoreMesh(core_axis_name="c", subcore_axis_name="s")


@functools.partial(
    pl.kernel,
    out_type=jax.ShapeDtypeStruct((TOKENS, EMBED_DIM), jnp.float32),
    mesh=_mesh,
    scratch_types=[
        pltpu.VMEM((CHUNKS_PER_WORKER, CHUNK), jnp.int32),
        pltpu.VMEM((NBUF, CHUNK, EMBED_DIM), jnp.float32),
    ] + [pltpu.SemaphoreType.DMA] * (2 * NBUF),
)
def _embedding_gather(idx_hbm, table_hbm, out_hbm, idx_v, rows_v, *sems):
    gsem = sems[:NBUF]
    ssem = sems[NBUF:]
    wid = lax.axis_index("c") * NUM_SUBCORES + lax.axis_index("s")
    base = wid * ROWS_PER_WORKER
    pltpu.sync_copy(idx_hbm.at[wid], idx_v)

    def start_gather(row, slot):
        pltpu.async_copy(table_hbm.at[idx_v.at[row]], rows_v.at[slot],
                         gsem[slot])

    def wait_gather(row, slot):
        pltpu.make_async_copy(table_hbm.at[idx_v.at[row]], rows_v.at[slot],
                              gsem[slot]).wait()

    def start_store(row, slot):
        pltpu.async_copy(rows_v.at[slot],
                         out_hbm.at[pl.ds(base + row * CHUNK, CHUNK)],
                         ssem[slot])

    def wait_store(row, slot):
        pltpu.make_async_copy(rows_v.at[slot],
                              out_hbm.at[pl.ds(base + row * CHUNK, CHUNK)],
                              ssem[slot]).wait()

    def full_step(t, b):
        g = t + LOOK
        gs = (b + LOOK) % NBUF
        wait_store(g - NBUF, gs)
        start_gather(g, gs)
        wait_gather(t, b)
        start_store(t, b)

    for b in range(LOOK):
        start_gather(b, b)

    for b in range(NBUF):
        g = b + LOOK
        gs = (b + LOOK) % NBUF
        if g >= NBUF:
            wait_store(g - NBUF, gs)
        start_gather(g, gs)
        wait_gather(b, b)
        start_store(b, b)

    def round_body(r, carry):
        for b in range(NBUF):
            full_step(r * NBUF + b, b)
        return carry

    lax.fori_loop(1, ROUNDS - 1, round_body, 0)

    r = ROUNDS - 1
    for b in range(NBUF):
        t = r * NBUF + b
        g = t + LOOK
        if g < CHUNKS_PER_WORKER:
            gs = (b + LOOK) % NBUF
            wait_store(g - NBUF, gs)
            start_gather(g, gs)
        wait_gather(t, b)
        start_store(t, b)

    for b in range(NBUF):
        wait_store(r * NBUF + b, b)


def kernel(token_ids, weight):
    flat = token_ids.reshape(-1).astype(jnp.int32)
    idx = flat.reshape(NUM_WORKERS, CHUNKS_PER_WORKER, CHUNK)
    out = _embedding_gather(idx, weight)
    return out.reshape(token_ids.shape + (weight.shape[1],))


# speedup vs baseline: 5.9753x; 1.7810x over previous
import functools

import jax
import jax.numpy as jnp
from jax import lax
from jax.experimental import pallas as pl
from jax.experimental.pallas import tpu as pltpu
from jax.experimental.pallas import tpu_sc as plsc

EMBED_DIM = 128
SEQ = 50
NROWS = 4096
NUM_CORES = 2
NUM_SUBCORES = 16
NUM_WORKERS = NUM_CORES * NUM_SUBCORES
ROWS_PER_WORKER = NROWS // NUM_WORKERS
NBUF = 8
LOOK = 6
ROUNDS = ROWS_PER_WORKER // NBUF

_mesh = plsc.VectorSubcoreMesh(core_axis_name="c", subcore_axis_name="s")


@functools.partial(
    pl.kernel,
    out_type=jax.ShapeDtypeStruct((NROWS, SEQ, EMBED_DIM), jnp.float32),
    mesh=_mesh,
    scratch_types=[
        pltpu.VMEM((ROWS_PER_WORKER, SEQ), jnp.int32),
        pltpu.VMEM((NBUF, SEQ, EMBED_DIM), jnp.float32),
    ] + [pltpu.SemaphoreType.DMA] * (2 * NBUF),
)
def _embedding_gather(idx_hbm, table_hbm, out_hbm, idx_v, rows_v, *sems):
    gsem = sems[:NBUF]
    ssem = sems[NBUF:]
    wid = lax.axis_index("c") * NUM_SUBCORES + lax.axis_index("s")
    base = wid * ROWS_PER_WORKER
    pltpu.sync_copy(idx_hbm.at[pl.ds(base, ROWS_PER_WORKER)], idx_v)

    def start_gather(row, slot):
        pltpu.async_copy(table_hbm.at[idx_v.at[row]], rows_v.at[slot],
                         gsem[slot])

    def wait_gather(row, slot):
        pltpu.make_async_copy(table_hbm.at[idx_v.at[row]], rows_v.at[slot],
                              gsem[slot]).wait()

    def start_store(row, slot):
        pltpu.async_copy(rows_v.at[slot], out_hbm.at[base + row], ssem[slot])

    def wait_store(row, slot):
        pltpu.make_async_copy(rows_v.at[slot], out_hbm.at[base + row],
                              ssem[slot]).wait()

    def full_step(t, b):
        g = t + LOOK
        gs = (b + LOOK) % NBUF
        wait_store(g - NBUF, gs)
        start_gather(g, gs)
        wait_gather(t, b)
        start_store(t, b)

    for b in range(LOOK):
        start_gather(b, b)

    for b in range(NBUF):
        g = b + LOOK
        gs = (b + LOOK) % NBUF
        if g >= NBUF:
            wait_store(g - NBUF, gs)
        start_gather(g, gs)
        wait_gather(b, b)
        start_store(b, b)

    def round_body(r, carry):
        for b in range(NBUF):
            full_step(r * NBUF + b, b)
        return carry

    lax.fori_loop(1, ROUNDS - 1, round_body, 0)

    r = ROUNDS - 1
    for b in range(NBUF):
        t = r * NBUF + b
        g = t + LOOK
        if g < ROWS_PER_WORKER:
            gs = (b + LOOK) % NBUF
            wait_store(g - NBUF, gs)
            start_gather(g, gs)
        wait_gather(t, b)
        start_store(t, b)

    for b in range(NBUF):
        wait_store(r * NBUF + b, b)


def kernel(token_ids, weight):
    return _embedding_gather(token_ids.astype(jnp.int32), weight)


# speedup vs baseline: 6.0131x; 1.0063x over previous
import functools

import jax
import jax.numpy as jnp
from jax import lax
from jax.experimental import pallas as pl
from jax.experimental.pallas import tpu as pltpu
from jax.experimental.pallas import tpu_sc as plsc

EMBED_DIM = 128
SEQ = 50
NROWS = 4096
NUM_CORES = 2
NUM_SUBCORES = 16
NUM_WORKERS = NUM_CORES * NUM_SUBCORES
ROWS_PER_WORKER = NROWS // NUM_WORKERS
PAIR = 2
IDX_W = PAIR * SEQ
STEPS = ROWS_PER_WORKER // PAIR
NBUF = 8
LOOK = 6
ROUNDS = STEPS // NBUF

_mesh = plsc.VectorSubcoreMesh(core_axis_name="c", subcore_axis_name="s")


@functools.partial(
    pl.kernel,
    out_type=jax.ShapeDtypeStruct((NROWS, SEQ, EMBED_DIM), jnp.float32),
    mesh=_mesh,
    scratch_types=[
        pltpu.VMEM((STEPS, IDX_W), jnp.int32),
        pltpu.VMEM((NBUF, IDX_W, EMBED_DIM), jnp.float32),
    ] + [pltpu.SemaphoreType.DMA] * (2 * NBUF),
)
def _embedding_gather(idx_hbm, table_hbm, out_hbm, idx_v, rows_v, *sems):
    gsem = sems[:NBUF]
    ssem = sems[NBUF:]
    wid = lax.axis_index("c") * NUM_SUBCORES + lax.axis_index("s")
    base = wid * ROWS_PER_WORKER
    pltpu.sync_copy(idx_hbm.at[pl.ds(wid * STEPS, STEPS)], idx_v)

    def start_gather(step, slot):
        pltpu.async_copy(table_hbm.at[idx_v.at[step]], rows_v.at[slot],
                         gsem[slot])

    def wait_gather(step, slot):
        pltpu.make_async_copy(table_hbm.at[idx_v.at[step]], rows_v.at[slot],
                              gsem[slot]).wait()

    def start_store(step, slot):
        for p in range(PAIR):
            pltpu.async_copy(rows_v.at[slot].at[pl.ds(p * SEQ, SEQ)],
                             out_hbm.at[base + step * PAIR + p],
                             ssem[slot])

    def wait_store(step, slot):
        for p in range(PAIR):
            pltpu.make_async_copy(rows_v.at[slot].at[pl.ds(p * SEQ, SEQ)],
                                  out_hbm.at[base + step * PAIR + p],
                                  ssem[slot]).wait()

    def full_step(t, b):
        g = t + LOOK
        gs = (b + LOOK) % NBUF
        wait_store(g - NBUF, gs)
        start_gather(g, gs)
        wait_gather(t, b)
        start_store(t, b)

    for b in range(LOOK):
        start_gather(b, b)

    for b in range(NBUF):
        g = b + LOOK
        gs = (b + LOOK) % NBUF
        if g >= NBUF:
            wait_store(g - NBUF, gs)
        start_gather(g, gs)
        wait_gather(b, b)
        start_store(b, b)

    def round_body(r, carry):
        for b in range(NBUF):
            full_step(r * NBUF + b, b)
        return carry

    lax.fori_loop(1, ROUNDS - 1, round_body, 0)

    r = ROUNDS - 1
    for b in range(NBUF):
        t = r * NBUF + b
        g = t + LOOK
        if g < STEPS:
            gs = (b + LOOK) % NBUF
            wait_store(g - NBUF, gs)
            start_gather(g, gs)
        wait_gather(t, b)
        start_store(t, b)

    for b in range(NBUF):
        wait_store(r * NBUF + b, b)


def kernel(token_ids, weight):
    idx = token_ids.astype(jnp.int32).reshape(NROWS // PAIR, IDX_W)
    return _embedding_gather(idx, weight)


# speedup vs baseline: 6.0284x; 1.0025x over previous
import functools

import jax
import jax.numpy as jnp
from jax import lax
from jax.experimental import pallas as pl
from jax.experimental.pallas import tpu as pltpu
from jax.experimental.pallas import tpu_sc as plsc

EMBED_DIM = 128
SEQ = 50
NROWS = 4096
NUM_CORES = 2
NUM_SUBCORES = 16
NUM_WORKERS = NUM_CORES * NUM_SUBCORES
ROWS_PER_WORKER = NROWS // NUM_WORKERS
PAIR = 2
IDX_W = PAIR * SEQ
STEPS = ROWS_PER_WORKER // PAIR
NBUF = 8
LOOK = 6
ROUNDS = STEPS // NBUF

_mesh = plsc.VectorSubcoreMesh(core_axis_name="c", subcore_axis_name="s")


@functools.partial(
    pl.kernel,
    out_type=jax.ShapeDtypeStruct((NROWS, SEQ, EMBED_DIM), jnp.float32),
    mesh=_mesh,
    compiler_params=pltpu.CompilerParams(use_tc_tiling_on_sc=True),
    scratch_types=[
        pltpu.VMEM((STEPS, IDX_W), jnp.int32),
        pltpu.VMEM((NBUF, IDX_W, EMBED_DIM), jnp.float32),
    ] + [pltpu.SemaphoreType.DMA] * (2 * NBUF),
)
def _embedding_gather(idx_hbm, table_hbm, out_hbm, idx_v, rows_v, *sems):
    gsem = sems[:NBUF]
    ssem = sems[NBUF:]
    wid = lax.axis_index("c") * NUM_SUBCORES + lax.axis_index("s")
    base = wid * ROWS_PER_WORKER
    pltpu.sync_copy(idx_hbm.at[pl.ds(wid * STEPS, STEPS)], idx_v)

    def start_gather(step, slot):
        pltpu.async_copy(table_hbm.at[idx_v.at[step]], rows_v.at[slot],
                         gsem[slot])

    def wait_gather(step, slot):
        pltpu.make_async_copy(table_hbm.at[idx_v.at[step]], rows_v.at[slot],
                              gsem[slot]).wait()

    def start_store(step, slot):
        for p in range(PAIR):
            pltpu.async_copy(rows_v.at[slot].at[pl.ds(p * SEQ, SEQ)],
                             out_hbm.at[base + step * PAIR + p],
                             ssem[slot])

    def wait_store(step, slot):
        for p in range(PAIR):
            pltpu.make_async_copy(rows_v.at[slot].at[pl.ds(p * SEQ, SEQ)],
                                  out_hbm.at[base + step * PAIR + p],
                                  ssem[slot]).wait()

    def full_step(t, b):
        g = t + LOOK
        gs = (b + LOOK) % NBUF
        wait_store(g - NBUF, gs)
        start_gather(g, gs)
        wait_gather(t, b)
        start_store(t, b)

    for b in range(LOOK):
        start_gather(b, b)

    for b in range(NBUF):
        g = b + LOOK
        gs = (b + LOOK) % NBUF
        if g >= NBUF:
            wait_store(g - NBUF, gs)
        start_gather(g, gs)
        wait_gather(b, b)
        start_store(b, b)

    def round_body(r, carry):
        for b in range(NBUF):
            full_step(r * NBUF + b, b)
        return carry

    lax.fori_loop(1, ROUNDS - 1, round_body, 0)

    r = ROUNDS - 1
    for b in range(NBUF):
        t = r * NBUF + b
        g = t + LOOK
        if g < STEPS:
            gs = (b + LOOK) % NBUF
            wait_store(g - NBUF, gs)
            start_gather(g, gs)
        wait_gather(t, b)
        start_store(t, b)

    for b in range(NBUF):
        wait_store(r * NBUF + b, b)


def kernel(token_ids, weight):
    idx = token_ids.astype(jnp.int32).reshape(NROWS // PAIR, IDX_W)
    return _embedding_gather(idx, weight)
